# speedup vs baseline: 1.5786x; 1.5786x over previous
import functools

import jax
import jax.numpy as jnp
from jax import lax
from jax.experimental import pallas as pl
from jax.experimental.pallas import tpu as pltpu
from jax.experimental.pallas import tpu_sc as plsc

_B, _L, _D = 4, 4096, 1024
_NC, _NS = 2, 16
_NW = _NC * _NS
_ROWS_PER_W = _L // _NW
_CHUNK = 64


def _make_pe_kernel():
    mesh = plsc.VectorSubcoreMesh(core_axis_name="c", subcore_axis_name="s")

    @functools.partial(
        pl.kernel,
        out_type=jax.ShapeDtypeStruct((_B, _L, _D), jnp.float32),
        mesh=mesh,
        scratch_types=[
            pltpu.VMEM((_CHUNK, _D), jnp.float32),
        ],
    )
    def pe_kernel(table_hbm, out_hbm, buf):
        wid = lax.axis_index("s") * _NC + lax.axis_index("c")
        base = wid * _ROWS_PER_W
        for c in range(_ROWS_PER_W // _CHUNK):
            start = base + c * _CHUNK
            pltpu.sync_copy(table_hbm.at[pl.ds(start, _CHUNK)], buf)
            for b in range(_B):
                pltpu.sync_copy(buf, out_hbm.at[b, pl.ds(start, _CHUNK)])

    return pe_kernel


_pe = _make_pe_kernel()


def kernel(seq_h, pos_table):
    del seq_h
    return _pe(pos_table)
